# Initial kernel scaffold; baseline (speedup 1.0000x reference)
import jax
import jax.numpy as jnp
from jax.experimental import pallas as pl


def kernel(x, weight, bias, gamma):
    raise NotImplementedError("write your pallas kernel here")


# speedup vs baseline: 1.0094x; 1.0094x over previous
import functools

import jax
import jax.numpy as jnp
from jax.experimental import pallas as pl
from jax.experimental.pallas import tpu as pltpu

B, C, H, W, K = 4, 192, 48, 48, 9
HW = H * W
RB = 256
NRB = HW // RB


def _diag_body(qc_ref, d2_ref):
    blk = qc_ref[0]
    m = jax.lax.dot_general(blk, blk, (((0,), (0,)), ((), ())),
                            preferred_element_type=jnp.float32)
    ii = jax.lax.broadcasted_iota(jnp.int32, (RB, RB), 0)
    jj = jax.lax.broadcasted_iota(jnp.int32, (RB, RB), 1)
    d2 = jnp.sum(jnp.where(ii == jj, m, 0.0), axis=1)
    d2_ref[0, 0] = d2


def _en_body(q_ref, d_ref, en_ref):
    j = pl.program_id(1)
    q = q_ref[0]
    a = q_ref[0, :, pl.ds(j * RB, RB)]
    e = jax.lax.dot_general(a, q, (((0,), (0,)), ((), ())),
                            preferred_element_type=jnp.float32)
    d = d_ref[0, 0]
    di = d_ref[0, 0, pl.ds(j * RB, RB)]
    en_ref[0] = (e / di[:, None]) / d[None, :]


def _compute_en(q):
    d2 = pl.pallas_call(
        _diag_body,
        grid=(B, NRB),
        in_specs=[pl.BlockSpec((1, C, RB), lambda b, j: (b, 0, j))],
        out_specs=pl.BlockSpec((1, 1, RB), lambda b, j: (b * NRB + j, 0, 0)),
        out_shape=jax.ShapeDtypeStruct((B * NRB, 1, RB), jnp.float32),
    )(q)
    d = jnp.sqrt(d2).reshape(B, 1, HW)
    en = pl.pallas_call(
        _en_body,
        grid=(B, NRB),
        in_specs=[
            pl.BlockSpec((1, C, HW), lambda b, j: (b, 0, 0)),
            pl.BlockSpec((1, 1, HW), lambda b, j: (b, 0, 0)),
        ],
        out_specs=pl.BlockSpec((1, RB, HW), lambda b, j: (b, j, 0)),
        out_shape=jax.ShapeDtypeStruct((B, HW, HW), jnp.float32),
    )(q, d)
    return en


def kernel(x, weight, bias, gamma):
    q = x.reshape(B, C, HW)
    en = _compute_en(q)
    idx_sorted = jnp.argsort(en, axis=-1)
    chunk = HW // K
    sel = idx_sorted[:, :, 0::chunk][:, :, :K]
    sel = jnp.sort(sel, axis=-1)
    gathered = jnp.take_along_axis(q[:, :, None, :], sel[:, None, :, :], axis=3)
    out = jnp.einsum('bcjk,ick->bij', gathered, weight) + bias[None, :, None]
    out = jax.nn.relu(out.reshape(B, C, H, W))
    return gamma.reshape(()) * out + x


# speedup vs baseline: 2.9394x; 2.9119x over previous
import functools

import jax
import jax.numpy as jnp
from jax import lax
from jax.experimental import pallas as pl
from jax.experimental.pallas import tpu as pltpu
from jax.experimental.pallas import tpu_sc as plsc

B, C, H, W, K = 4, 192, 48, 48, 9
HW = H * W
RB = 256
NRB = HW // RB
NB = 2048
NV = HW // 16
CHUNK = HW // K
NW = 32
RPW = HW // NW


def _diag_body(qc_ref, d2_ref):
    blk = qc_ref[0]
    m = lax.dot_general(blk, blk, (((0,), (0,)), ((), ())),
                        preferred_element_type=jnp.float32)
    ii = lax.broadcasted_iota(jnp.int32, (RB, RB), 0)
    jj = lax.broadcasted_iota(jnp.int32, (RB, RB), 1)
    d2_ref[0, 0] = jnp.sum(jnp.where(ii == jj, m, 0.0), axis=1)


def _en_body(q_ref, d_ref, en_ref):
    j = pl.program_id(1)
    q = q_ref[0]
    a = q_ref[0, :, pl.ds(j * RB, RB)]
    e = lax.dot_general(a, q, (((0,), (0,)), ((), ())),
                        preferred_element_type=jnp.float32)
    d = d_ref[0, 0]
    di = d_ref[0, 0, pl.ds(j * RB, RB)]
    en_ref[0] = (e / di[:, None]) / d[None, :]


def _compute_en(q):
    d2 = pl.pallas_call(
        _diag_body,
        grid=(B, NRB),
        in_specs=[pl.BlockSpec((1, C, RB), lambda b, j: (b, 0, j))],
        out_specs=pl.BlockSpec((1, 1, RB), lambda b, j: (b * NRB + j, 0, 0)),
        out_shape=jax.ShapeDtypeStruct((B * NRB, 1, RB), jnp.float32),
    )(q)
    d = jnp.sqrt(d2).reshape(B, 1, HW)
    en = pl.pallas_call(
        _en_body,
        grid=(B, NRB),
        in_specs=[
            pl.BlockSpec((1, C, HW), lambda b, j: (b, 0, 0)),
            pl.BlockSpec((1, 1, HW), lambda b, j: (b, 0, 0)),
        ],
        out_specs=pl.BlockSpec((1, RB, HW), lambda b, j: (b, j, 0)),
        out_shape=jax.ShapeDtypeStruct((B, HW, HW), jnp.float32),
    )(q, d)
    return en


def _yt_body(q_ref, w_ref, yt_ref):
    q = q_ref[0]
    wk = w_ref[0]
    yt_ref[0, 0] = lax.dot_general(q, wk, (((0,), (1,)), ((), ())),
                                   preferred_element_type=jnp.float32)


def _compute_yt(q, wt):
    return pl.pallas_call(
        _yt_body,
        grid=(B, K),
        in_specs=[
            pl.BlockSpec((1, C, HW), lambda b, k: (b, 0, 0)),
            pl.BlockSpec((1, C, C), lambda b, k: (k, 0, 0)),
        ],
        out_specs=pl.BlockSpec((1, 1, HW, C), lambda b, k: (b, k, 0, 0)),
        out_shape=jax.ShapeDtypeStruct((B, K, HW, C), jnp.float32),
    )(q, wt)


@functools.cache
def _make_sc_select():
    mesh = plsc.VectorSubcoreMesh(core_axis_name="c", subcore_axis_name="s")
    return pl.kernel(
        _sc_body,
        out_type=(
            jax.ShapeDtypeStruct((B * HW, C), jnp.float32),
            jax.ShapeDtypeStruct((B * HW, 16), jnp.int32),
        ),
        mesh=mesh,
        scratch_types=[
            pltpu.VMEM((HW,), jnp.float32),
            pltpu.VMEM((HW,), jnp.int32),
            pltpu.VMEM((NB,), jnp.float32),
            pltpu.VMEM((NB,), jnp.float32),
            pltpu.VMEM((NB,), jnp.int32),
            pltpu.VMEM((512,), jnp.float32),
            pltpu.VMEM((512,), jnp.int32),
            pltpu.VMEM((512,), jnp.int32),
            pltpu.VMEM((256,), jnp.float32),
            pltpu.VMEM((256,), jnp.int32),
            pltpu.VMEM((32,), jnp.int32),
            pltpu.VMEM((16,), jnp.float32),
            pltpu.VMEM((16,), jnp.int32),
            pltpu.VMEM((16,), jnp.float32),
            pltpu.VMEM((16,), jnp.int32),
            pltpu.VMEM((16, C), jnp.float32),
            pltpu.VMEM((C,), jnp.float32),
            pltpu.VMEM((16,), jnp.int32),
            pltpu.SemaphoreType.DMA,
        ],
        compiler_params=pltpu.CompilerParams(needs_layout_passes=False,
                                             use_tc_tiling_on_sc=False),
    )


def _sc_body(en_hbm, yt_hbm, acc_hbm, sel_hbm,
               vals, binb, hist, cumb, flag, fv, fi, fs, tbv, tbi, tie,
               scrf, scri, scrt, scrs, ytrows, accb, selbuf, sem):
    cid = lax.axis_index("c")
    sid = lax.axis_index("s")
    wid = sid * 2 + cid

    iota = lax.iota(jnp.int32, 16)
    zf = jnp.zeros((16,), jnp.float32)
    zi = jnp.zeros((16,), jnp.int32)
    onesf = jnp.ones((16,), jnp.float32)

    def zero_init(k2, c):
        hist[pl.ds(k2 * 16, 16)] = zf
        flag[pl.ds(k2 * 16, 16)] = zi
        return c
    lax.fori_loop(0, NB // 16, zero_init, 0)

    def row_body(b, r):
        row = b * HW + wid * RPW + r
        pltpu.sync_copy(en_hbm.at[row], vals)

        def hpass(i, c):
            v = vals[pl.ds(i * 16, 16)]
            bi = jnp.clip(((v + 0.5) * float(NB)).astype(jnp.int32), 0, NB - 1)
            binb[pl.ds(i * 16, 16)] = bi
            plsc.addupdate_scatter(hist, [bi], onesf)
            return c
        lax.fori_loop(0, NV, hpass, 0)

        def cpass(k2, off):
            h = hist[pl.ds(k2 * 16, 16)]
            cs = plsc.cumsum(h)
            cumb[pl.ds(k2 * 16, 16)] = off + (cs - h)
            return off + jnp.max(cs)
        lax.fori_loop(0, NB // 16, cpass, jnp.float32(0.0))

        bvec = zi
        tresv = zf
        p16 = [plsc.load_gather(cumb, [g * 256 + iota * 16]) for g in range(8)]
        for rk in range(K):
            t = jnp.float32(rk * CHUNK)
            c1 = zi
            for g in range(8):
                c1 = c1 + plsc.all_reduce_population_count(p16[g] <= t)
            v = c1 - 1
            cumv = plsc.load_gather(cumb, [v * 16 + iota])
            l = plsc.all_reduce_population_count(cumv <= t)
            b_r = v * 16 + l - 1
            prefb = plsc.load_gather(cumb, [b_r])
            bvec = jnp.where(iota == rk, b_r, bvec)
            tresv = jnp.where(iota == rk, t - prefb, tresv)

        m9 = iota < K
        plsc.store_scatter(flag, [bvec], iota + 1, mask=m9)
        slotv = plsc.load_gather(flag, [bvec])

        def epass(i, nf):
            bi = binb[pl.ds(i * 16, 16)]
            f = plsc.load_gather(flag, [bi])
            plsc.store_scatter(hist, [bi], zf)
            m = f > 0
            v = vals[pl.ds(i * 16, 16)]
            plsc.store_compressed(fv.at[pl.ds(nf, 16)], v, mask=m)
            plsc.store_compressed(fi.at[pl.ds(nf, 16)], i * 16 + iota, mask=m)
            plsc.store_compressed(fs.at[pl.ds(nf, 16)], f, mask=m)
            nf = nf + jnp.sum(m.astype(jnp.int32))
            return jnp.minimum(nf, 496)
        nf = lax.fori_loop(0, NV, epass, jnp.int32(0))
        plsc.store_scatter(flag, [bvec], zi, mask=m9)

        selv = zi
        nvf = (nf + 15) >> 4
        for rk in range(K):
            slot = jnp.sum(jnp.where(iota == rk, slotv, 0))
            tres = jnp.sum(jnp.where(iota == rk, tresv, 0.0)).astype(jnp.int32)

            def fscan(u, nb):
                fvv = fv[pl.ds(u * 16, 16)]
                fss = fs[pl.ds(u * 16, 16)]
                fii = fi[pl.ds(u * 16, 16)]
                m = (fss == slot) & ((u * 16 + iota) < nf)
                plsc.store_compressed(tbv.at[pl.ds(nb, 16)], fvv, mask=m)
                plsc.store_compressed(tbi.at[pl.ds(nb, 16)], fii, mask=m)
                nb = nb + jnp.sum(m.astype(jnp.int32))
                return jnp.minimum(nb, 240)
            nb = lax.fori_loop(0, nvf, fscan, jnp.int32(0))

            def path_a(_):
                tv = tbv[pl.ds(0, 16)]
                ti = tbi[pl.ds(0, 16)]
                lm = iota < nb
                sk, sv, _ = plsc.sort_key_val(tv, ti, mask=lm)
                vstar = jnp.sum(jnp.where(iota == tres, sk, 0.0))
                cl = plsc.all_reduce_population_count((tv < vstar) & lm)
                tiem = (tv == vstar) & lm
                si, _, _ = plsc.sort_key_val(ti, ti, mask=tiem)
                pick = jnp.sum(jnp.where(iota == tres - cl, si, 0))
                return jnp.full((16,), pick, jnp.int32)

            def path_b(_):
                nbv = (nb + 15) >> 4

                def skey(x):
                    s = plsc.bitcast(x, jnp.int32)
                    return plsc.bitcast(
                        jnp.where(s < 0, ~s, s | jnp.int32(-2147483648)),
                        jnp.uint32)

                def bit_step(it, lo):
                    cand = lo | (jnp.uint32(1) << (31 - it).astype(jnp.uint32))

                    def cnt_scan(u, c):
                        kv = skey(tbv[pl.ds(u * 16, 16)])
                        m = (kv < cand) & ((u * 16 + iota) < nb)
                        return c + jnp.sum(m.astype(jnp.int32))
                    cnt = lax.fori_loop(0, nbv, cnt_scan, jnp.int32(0))
                    return jnp.where(cnt <= tres, cand, lo)
                lo = lax.fori_loop(0, 32, bit_step, jnp.uint32(0))
                si32 = plsc.bitcast(jnp.full((16,), lo, jnp.uint32), jnp.int32)
                vbits = jnp.where(si32 < 0, si32 & jnp.int32(2147483647), ~si32)
                vstar = plsc.bitcast(vbits, jnp.float32)

                def tie_scan(u, carry):
                    cl, nt = carry
                    tv = tbv[pl.ds(u * 16, 16)]
                    ti2 = tbi[pl.ds(u * 16, 16)]
                    valid = (u * 16 + iota) < nb
                    cl = cl + jnp.sum(((tv < vstar) & valid).astype(jnp.int32))
                    tm = (tv == vstar) & valid
                    plsc.store_compressed(tie.at[pl.ds(nt, 16)], ti2, mask=tm)
                    nt = jnp.minimum(nt + jnp.sum(tm.astype(jnp.int32)), 16)
                    return (cl, nt)
                cl, nt = lax.fori_loop(0, nbv, tie_scan,
                                       (jnp.int32(0), jnp.int32(0)))
                tv16 = tie[pl.ds(0, 16)]
                si, _, _ = plsc.sort_key_val(tv16, tv16, mask=iota < nt)
                pick = jnp.sum(jnp.where(iota == tres - cl, si, 0))
                return jnp.full((16,), pick, jnp.int32)

            pick = lax.cond(nb <= 16, path_a, path_b, 0)
            selv = jnp.where(iota == rk, pick, selv)

        ssel, _, _ = plsc.sort_key_val(selv, selv, mask=m9)
        selbuf[...] = jnp.where(m9, ssel, 0)
        pltpu.sync_copy(selbuf, sel_hbm.at[row])
        ytidx = jnp.where(m9, (b * K + iota) * HW + ssel, 0)
        pltpu.async_copy(yt_hbm.at[ytidx], ytrows, sem).wait()
        for cb in range(C // 16):
            a = ytrows[0, pl.ds(cb * 16, 16)]
            for k in range(1, K):
                a = a + ytrows[k, pl.ds(cb * 16, 16)]
            accb[pl.ds(cb * 16, 16)] = a
        pltpu.sync_copy(accb, acc_hbm.at[row])
        return r

    def batch_body(b, c):
        def rloop(r, c2):
            row_body(b, r)
            return c2
        lax.fori_loop(0, RPW, rloop, 0)
        return c
    lax.fori_loop(0, B, batch_body, 0)


def _ep_body(acc_ref, bias_ref, x_ref, g_ref, out_ref):
    a = acc_ref[0]
    t = a + bias_ref[0][None, :]
    r = jnp.maximum(t, 0.0)
    rt = r.T
    out_ref[0] = g_ref[0, 0] * rt + x_ref[0]


def _epilogue(acc, bias, x3, gamma):
    return pl.pallas_call(
        _ep_body,
        grid=(B, NRB),
        in_specs=[
            pl.BlockSpec((1, RB, C), lambda b, j: (b * NRB + j, 0, 0)),
            pl.BlockSpec((1, C), lambda b, j: (0, 0)),
            pl.BlockSpec((1, C, RB), lambda b, j: (b, 0, j)),
            pl.BlockSpec((1, 1), lambda b, j: (0, 0)),
        ],
        out_specs=pl.BlockSpec((1, C, RB), lambda b, j: (b, 0, j)),
        out_shape=jax.ShapeDtypeStruct((B, C, HW), jnp.float32),
    )(acc, bias, x3, gamma)


def kernel(x, weight, bias, gamma):
    q = x.reshape(B, C, HW)
    en = _compute_en(q)
    yt = _compute_yt(q, jnp.transpose(weight, (2, 0, 1)))
    acc, _sel = _make_sc_select()(en.reshape(B * HW, HW),
                                  yt.reshape(B * K * HW, C))
    out = _epilogue(acc.reshape(B * NRB, RB, C), bias.reshape(1, C), q,
                    gamma.reshape(1, 1))
    return out.reshape(B, C, H, W)


# speedup vs baseline: 2.9899x; 1.0172x over previous
import functools

import jax
import jax.numpy as jnp
from jax import lax
from jax.experimental import pallas as pl
from jax.experimental.pallas import tpu as pltpu
from jax.experimental.pallas import tpu_sc as plsc

B, C, H, W, K = 4, 192, 48, 48, 9
HW = H * W
RB = 256
NRB = HW // RB
NB = 2048
NV = HW // 16
CHUNK = HW // K
NW = 32
RPW = HW // NW


def _diag_body(qc_ref, d2_ref):
    blk = qc_ref[0]
    m = lax.dot_general(blk, blk, (((0,), (0,)), ((), ())),
                        preferred_element_type=jnp.float32)
    ii = lax.broadcasted_iota(jnp.int32, (RB, RB), 0)
    jj = lax.broadcasted_iota(jnp.int32, (RB, RB), 1)
    d2_ref[0, 0] = jnp.sum(jnp.where(ii == jj, m, 0.0), axis=1)


def _en_body(q_ref, d_ref, en_ref):
    j = pl.program_id(1)
    q = q_ref[0]
    a = q_ref[0, :, pl.ds(j * RB, RB)]
    e = lax.dot_general(a, q, (((0,), (0,)), ((), ())),
                        preferred_element_type=jnp.float32)
    d = d_ref[0, 0]
    di = d_ref[0, 0, pl.ds(j * RB, RB)]
    en_ref[0] = (e / di[:, None]) / d[None, :]


def _compute_en(q):
    d2 = pl.pallas_call(
        _diag_body,
        grid=(B, NRB),
        in_specs=[pl.BlockSpec((1, C, RB), lambda b, j: (b, 0, j))],
        out_specs=pl.BlockSpec((1, 1, RB), lambda b, j: (b * NRB + j, 0, 0)),
        out_shape=jax.ShapeDtypeStruct((B * NRB, 1, RB), jnp.float32),
    )(q)
    d = jnp.sqrt(d2).reshape(B, 1, HW)
    en = pl.pallas_call(
        _en_body,
        grid=(B, NRB),
        in_specs=[
            pl.BlockSpec((1, C, HW), lambda b, j: (b, 0, 0)),
            pl.BlockSpec((1, 1, HW), lambda b, j: (b, 0, 0)),
        ],
        out_specs=pl.BlockSpec((1, RB, HW), lambda b, j: (b, j, 0)),
        out_shape=jax.ShapeDtypeStruct((B, HW, HW), jnp.float32),
    )(q, d)
    return en


def _yt_body(q_ref, w_ref, yt_ref):
    q = q_ref[0]
    wk = w_ref[0]
    yt_ref[0, 0] = lax.dot_general(q, wk, (((0,), (1,)), ((), ())),
                                   preferred_element_type=jnp.float32)


def _compute_yt(q, wt):
    return pl.pallas_call(
        _yt_body,
        grid=(B, K),
        in_specs=[
            pl.BlockSpec((1, C, HW), lambda b, k: (b, 0, 0)),
            pl.BlockSpec((1, C, C), lambda b, k: (k, 0, 0)),
        ],
        out_specs=pl.BlockSpec((1, 1, HW, C), lambda b, k: (b, k, 0, 0)),
        out_shape=jax.ShapeDtypeStruct((B, K, HW, C), jnp.float32),
    )(q, wt)


@functools.cache
def _make_sc_select():
    mesh = plsc.VectorSubcoreMesh(core_axis_name="c", subcore_axis_name="s")
    return pl.kernel(
        _sc_body,
        out_type=(
            jax.ShapeDtypeStruct((B * HW, C), jnp.float32),
            jax.ShapeDtypeStruct((B * HW, 16), jnp.int32),
        ),
        mesh=mesh,
        scratch_types=[
            pltpu.VMEM((HW,), jnp.float32),
            pltpu.VMEM((HW,), jnp.int32),
            pltpu.VMEM((NB,), jnp.float32),
            pltpu.VMEM((NB,), jnp.float32),
            pltpu.VMEM((NB,), jnp.int32),
            pltpu.VMEM((512,), jnp.float32),
            pltpu.VMEM((512,), jnp.int32),
            pltpu.VMEM((512,), jnp.int32),
            pltpu.VMEM((256,), jnp.float32),
            pltpu.VMEM((256,), jnp.int32),
            pltpu.VMEM((32,), jnp.int32),
            pltpu.VMEM((16,), jnp.float32),
            pltpu.VMEM((16,), jnp.int32),
            pltpu.VMEM((16,), jnp.float32),
            pltpu.VMEM((16,), jnp.int32),
            pltpu.VMEM((16, C), jnp.float32),
            pltpu.VMEM((C,), jnp.float32),
            pltpu.VMEM((16,), jnp.int32),
            pltpu.SemaphoreType.DMA,
        ],
        compiler_params=pltpu.CompilerParams(needs_layout_passes=False,
                                             use_tc_tiling_on_sc=False),
    )


def _sc_body(en_hbm, yt_hbm, acc_hbm, sel_hbm,
               vals, binb, hist, cumb, flag, fv, fi, fs, tbv, tbi, tie,
               scrf, scri, scrt, scrs, ytrows, accb, selbuf, sem):
    cid = lax.axis_index("c")
    sid = lax.axis_index("s")
    wid = sid * 2 + cid

    iota = lax.iota(jnp.int32, 16)
    zf = jnp.zeros((16,), jnp.float32)
    zi = jnp.zeros((16,), jnp.int32)
    onesf = jnp.ones((16,), jnp.float32)

    def zero_init(k2, c):
        hist[pl.ds(k2 * 16, 16)] = zf
        flag[pl.ds(k2 * 16, 16)] = zi
        return c
    lax.fori_loop(0, NB // 16, zero_init, 0)

    def row_body(b, r):
        row = b * HW + wid * RPW + r
        pltpu.sync_copy(en_hbm.at[row], vals)

        @plsc.parallel_loop(0, NV, unroll=4)
        def hpass(i):
            v = vals[pl.ds(i * 16, 16)]
            bi = jnp.clip(((v + 0.5) * float(NB)).astype(jnp.int32), 0, NB - 1)
            binb[pl.ds(i * 16, 16)] = bi
            plsc.addupdate_scatter(hist, [bi], onesf)

        def cpass(k2, off):
            h = hist[pl.ds(k2 * 16, 16)]
            cs = plsc.cumsum(h)
            cumb[pl.ds(k2 * 16, 16)] = off + (cs - h)
            return off + jnp.max(cs)
        lax.fori_loop(0, NB // 16, cpass, jnp.float32(0.0))

        bvec = zi
        tresv = zf
        p16 = [plsc.load_gather(cumb, [g * 256 + iota * 16]) for g in range(8)]
        for rk in range(K):
            t = jnp.float32(rk * CHUNK)
            c1 = zi
            for g in range(8):
                c1 = c1 + plsc.all_reduce_population_count(p16[g] <= t)
            v = c1 - 1
            cumv = plsc.load_gather(cumb, [v * 16 + iota])
            l = plsc.all_reduce_population_count(cumv <= t)
            b_r = v * 16 + l - 1
            prefb = plsc.load_gather(cumb, [b_r])
            bvec = jnp.where(iota == rk, b_r, bvec)
            tresv = jnp.where(iota == rk, t - prefb, tresv)

        m9 = iota < K
        plsc.store_scatter(flag, [bvec], iota + 1, mask=m9)
        slotv = plsc.load_gather(flag, [bvec])

        def epass(i4, nf):
            i = i4 * 4
            parts = []
            for dd in range(4):
                bi = binb[pl.ds((i + dd) * 16, 16)]
                f = plsc.load_gather(flag, [bi])
                plsc.store_scatter(hist, [bi], zf)
                m = f > 0
                v = vals[pl.ds((i + dd) * 16, 16)]
                packed = ((i + dd) * 16 + iota) | (f << 12)
                parts.append((m, v, packed, jnp.sum(m.astype(jnp.int32))))
            off = nf
            for (m, v, packed, cnt) in parts:
                off = jnp.minimum(off, 496)
                plsc.store_compressed(fv.at[pl.ds(off, 16)], v, mask=m)
                plsc.store_compressed(fi.at[pl.ds(off, 16)], packed, mask=m)
                off = off + cnt
            return jnp.minimum(off, 496)
        nf = lax.fori_loop(0, NV // 4, epass, jnp.int32(0))
        plsc.store_scatter(flag, [bvec], zi, mask=m9)

        selv = zi
        nvf = (nf + 15) >> 4
        for rk in range(K):
            slot = jnp.sum(jnp.where(iota == rk, slotv, 0))
            tres = jnp.sum(jnp.where(iota == rk, tresv, 0.0)).astype(jnp.int32)

            def fscan(u, nb):
                fvv = fv[pl.ds(u * 16, 16)]
                fpp = fi[pl.ds(u * 16, 16)]
                m = ((fpp >> 12) == slot) & ((u * 16 + iota) < nf)
                plsc.store_compressed(tbv.at[pl.ds(nb, 16)], fvv, mask=m)
                plsc.store_compressed(tbi.at[pl.ds(nb, 16)], fpp & 0xFFF,
                                      mask=m)
                nb = nb + jnp.sum(m.astype(jnp.int32))
                return jnp.minimum(nb, 240)
            nb = lax.fori_loop(0, nvf, fscan, jnp.int32(0))

            def path_a(_):
                tv = tbv[pl.ds(0, 16)]
                ti = tbi[pl.ds(0, 16)]
                lm = iota < nb
                sk, sv, _ = plsc.sort_key_val(tv, ti, mask=lm)
                vstar = jnp.sum(jnp.where(iota == tres, sk, 0.0))
                cl = plsc.all_reduce_population_count((tv < vstar) & lm)
                tiem = (tv == vstar) & lm
                si, _, _ = plsc.sort_key_val(ti, ti, mask=tiem)
                pick = jnp.sum(jnp.where(iota == tres - cl, si, 0))
                return jnp.full((16,), pick, jnp.int32)

            def path_b(_):
                nbv = (nb + 15) >> 4

                def skey(x):
                    s = plsc.bitcast(x, jnp.int32)
                    return plsc.bitcast(
                        jnp.where(s < 0, ~s, s | jnp.int32(-2147483648)),
                        jnp.uint32)

                def bit_step(it, lo):
                    cand = lo | (jnp.uint32(1) << (31 - it).astype(jnp.uint32))

                    def cnt_scan(u, c):
                        kv = skey(tbv[pl.ds(u * 16, 16)])
                        m = (kv < cand) & ((u * 16 + iota) < nb)
                        return c + jnp.sum(m.astype(jnp.int32))
                    cnt = lax.fori_loop(0, nbv, cnt_scan, jnp.int32(0))
                    return jnp.where(cnt <= tres, cand, lo)
                lo = lax.fori_loop(0, 32, bit_step, jnp.uint32(0))
                si32 = plsc.bitcast(jnp.full((16,), lo, jnp.uint32), jnp.int32)
                vbits = jnp.where(si32 < 0, si32 & jnp.int32(2147483647), ~si32)
                vstar = plsc.bitcast(vbits, jnp.float32)

                def tie_scan(u, carry):
                    cl, nt = carry
                    tv = tbv[pl.ds(u * 16, 16)]
                    ti2 = tbi[pl.ds(u * 16, 16)]
                    valid = (u * 16 + iota) < nb
                    cl = cl + jnp.sum(((tv < vstar) & valid).astype(jnp.int32))
                    tm = (tv == vstar) & valid
                    plsc.store_compressed(tie.at[pl.ds(nt, 16)], ti2, mask=tm)
                    nt = jnp.minimum(nt + jnp.sum(tm.astype(jnp.int32)), 16)
                    return (cl, nt)
                cl, nt = lax.fori_loop(0, nbv, tie_scan,
                                       (jnp.int32(0), jnp.int32(0)))
                tv16 = tie[pl.ds(0, 16)]
                si, _, _ = plsc.sort_key_val(tv16, tv16, mask=iota < nt)
                pick = jnp.sum(jnp.where(iota == tres - cl, si, 0))
                return jnp.full((16,), pick, jnp.int32)

            pick = lax.cond(nb <= 16, path_a, path_b, 0)
            selv = jnp.where(iota == rk, pick, selv)

        ssel, _, _ = plsc.sort_key_val(selv, selv, mask=m9)
        selbuf[...] = jnp.where(m9, ssel, 0)
        pltpu.sync_copy(selbuf, sel_hbm.at[row])
        ytidx = jnp.where(m9, (b * K + iota) * HW + ssel, 0)
        pltpu.async_copy(yt_hbm.at[ytidx], ytrows, sem).wait()
        for cb in range(C // 16):
            a = ytrows[0, pl.ds(cb * 16, 16)]
            for k in range(1, K):
                a = a + ytrows[k, pl.ds(cb * 16, 16)]
            accb[pl.ds(cb * 16, 16)] = a
        pltpu.sync_copy(accb, acc_hbm.at[row])
        return r

    def batch_body(b, c):
        def rloop(r, c2):
            row_body(b, r)
            return c2
        lax.fori_loop(0, RPW, rloop, 0)
        return c
    lax.fori_loop(0, B, batch_body, 0)


def _ep_body(acc_ref, bias_ref, x_ref, g_ref, out_ref):
    a = acc_ref[0]
    t = a + bias_ref[0][None, :]
    r = jnp.maximum(t, 0.0)
    rt = r.T
    out_ref[0] = g_ref[0, 0] * rt + x_ref[0]


def _epilogue(acc, bias, x3, gamma):
    return pl.pallas_call(
        _ep_body,
        grid=(B, NRB),
        in_specs=[
            pl.BlockSpec((1, RB, C), lambda b, j: (b * NRB + j, 0, 0)),
            pl.BlockSpec((1, C), lambda b, j: (0, 0)),
            pl.BlockSpec((1, C, RB), lambda b, j: (b, 0, j)),
            pl.BlockSpec((1, 1), lambda b, j: (0, 0)),
        ],
        out_specs=pl.BlockSpec((1, C, RB), lambda b, j: (b, 0, j)),
        out_shape=jax.ShapeDtypeStruct((B, C, HW), jnp.float32),
    )(acc, bias, x3, gamma)


def kernel(x, weight, bias, gamma):
    q = x.reshape(B, C, HW)
    en = _compute_en(q)
    yt = _compute_yt(q, jnp.transpose(weight, (2, 0, 1)))
    acc, _sel = _make_sc_select()(en.reshape(B * HW, HW),
                                  yt.reshape(B * K * HW, C))
    out = _epilogue(acc.reshape(B * NRB, RB, C), bias.reshape(1, C), q,
                    gamma.reshape(1, 1))
    return out.reshape(B, C, H, W)


# speedup vs baseline: 3.0001x; 1.0034x over previous
import functools

import jax
import jax.numpy as jnp
from jax import lax
from jax.experimental import pallas as pl
from jax.experimental.pallas import tpu as pltpu
from jax.experimental.pallas import tpu_sc as plsc

B, C, H, W, K = 4, 192, 48, 48, 9
HW = H * W
RB = 256
NRB = HW // RB
NB = 2048
NV = HW // 16
CHUNK = HW // K
NW = 32
RPW = HW // NW


def _diag_body(qc_ref, d2_ref):
    blk = qc_ref[0]
    m = lax.dot_general(blk, blk, (((0,), (0,)), ((), ())),
                        preferred_element_type=jnp.float32)
    ii = lax.broadcasted_iota(jnp.int32, (RB, RB), 0)
    jj = lax.broadcasted_iota(jnp.int32, (RB, RB), 1)
    d2_ref[0, 0] = jnp.sum(jnp.where(ii == jj, m, 0.0), axis=1)


def _en_body(q_ref, d_ref, en_ref):
    j = pl.program_id(1)
    q = q_ref[0]
    a = q_ref[0, :, pl.ds(j * RB, RB)]
    e = lax.dot_general(a, q, (((0,), (0,)), ((), ())),
                        preferred_element_type=jnp.float32)
    d = d_ref[0, 0]
    di = d_ref[0, 0, pl.ds(j * RB, RB)]
    en_ref[0] = (e / di[:, None]) / d[None, :]


def _compute_en(q):
    d2 = pl.pallas_call(
        _diag_body,
        grid=(B, NRB),
        in_specs=[pl.BlockSpec((1, C, RB), lambda b, j: (b, 0, j))],
        out_specs=pl.BlockSpec((1, 1, RB), lambda b, j: (b * NRB + j, 0, 0)),
        out_shape=jax.ShapeDtypeStruct((B * NRB, 1, RB), jnp.float32),
    )(q)
    d = jnp.sqrt(d2).reshape(B, 1, HW)
    en = pl.pallas_call(
        _en_body,
        grid=(B, NRB),
        in_specs=[
            pl.BlockSpec((1, C, HW), lambda b, j: (b, 0, 0)),
            pl.BlockSpec((1, 1, HW), lambda b, j: (b, 0, 0)),
        ],
        out_specs=pl.BlockSpec((1, RB, HW), lambda b, j: (b, j, 0)),
        out_shape=jax.ShapeDtypeStruct((B, HW, HW), jnp.float32),
    )(q, d)
    return en


def _yt_body(q_ref, w_ref, yt_ref):
    q = q_ref[0]
    wk = w_ref[0]
    yt_ref[0, 0] = lax.dot_general(q, wk, (((0,), (1,)), ((), ())),
                                   preferred_element_type=jnp.float32)


def _compute_yt(q, wt):
    return pl.pallas_call(
        _yt_body,
        grid=(B, K),
        in_specs=[
            pl.BlockSpec((1, C, HW), lambda b, k: (b, 0, 0)),
            pl.BlockSpec((1, C, C), lambda b, k: (k, 0, 0)),
        ],
        out_specs=pl.BlockSpec((1, 1, HW, C), lambda b, k: (b, k, 0, 0)),
        out_shape=jax.ShapeDtypeStruct((B, K, HW, C), jnp.float32),
    )(q, wt)


@functools.cache
def _make_sc_select():
    mesh = plsc.VectorSubcoreMesh(core_axis_name="c", subcore_axis_name="s")
    return pl.kernel(
        _sc_body,
        out_type=jax.ShapeDtypeStruct((B * HW, C), jnp.float32),
        mesh=mesh,
        scratch_types=[
            pltpu.VMEM((HW,), jnp.float32),
            pltpu.VMEM((HW,), jnp.int32),
            pltpu.VMEM((NB,), jnp.float32),
            pltpu.VMEM((NB,), jnp.float32),
            pltpu.VMEM((NB,), jnp.int32),
            pltpu.VMEM((512,), jnp.float32),
            pltpu.VMEM((512,), jnp.int32),
            pltpu.VMEM((256,), jnp.float32),
            pltpu.VMEM((256,), jnp.int32),
            pltpu.VMEM((32,), jnp.int32),
            pltpu.VMEM((16, C), jnp.float32),
            pltpu.VMEM((C,), jnp.float32),
            pltpu.SemaphoreType.DMA,
        ],
        compiler_params=pltpu.CompilerParams(needs_layout_passes=False,
                                             use_tc_tiling_on_sc=False),
    )


def _sc_body(en_hbm, yt_hbm, acc_hbm,
             vals, binb, hist, cumb, flag, fv, fi, tbv, tbi, tie,
             ytrows, accb, sem):
    cid = lax.axis_index("c")
    sid = lax.axis_index("s")
    wid = sid * 2 + cid

    iota = lax.iota(jnp.int32, 16)
    zf = jnp.zeros((16,), jnp.float32)
    zi = jnp.zeros((16,), jnp.int32)
    onesf = jnp.ones((16,), jnp.float32)

    def zero_init(k2, c):
        hist[pl.ds(k2 * 16, 16)] = zf
        flag[pl.ds(k2 * 16, 16)] = zi
        return c
    lax.fori_loop(0, NB // 16, zero_init, 0)

    def row_body(b, r):
        row = b * HW + wid * RPW + r
        pltpu.sync_copy(en_hbm.at[row], vals)

        @plsc.parallel_loop(0, NV, unroll=4)
        def hpass(i):
            v = vals[pl.ds(i * 16, 16)]
            bi = jnp.clip(((v + 0.5) * float(NB)).astype(jnp.int32), 0, NB - 1)
            binb[pl.ds(i * 16, 16)] = bi
            plsc.addupdate_scatter(hist, [bi], onesf)

        def cpass(k2, off):
            h = hist[pl.ds(k2 * 16, 16)]
            cs = plsc.cumsum(h)
            cumb[pl.ds(k2 * 16, 16)] = off + (cs - h)
            return off + jnp.max(cs)
        lax.fori_loop(0, NB // 16, cpass, jnp.float32(0.0))

        bvec = zi
        tresv = zf
        p16 = [plsc.load_gather(cumb, [g * 256 + iota * 16]) for g in range(8)]
        for rk in range(K):
            t = jnp.float32(rk * CHUNK)
            c1 = zi
            for g in range(8):
                c1 = c1 + plsc.all_reduce_population_count(p16[g] <= t)
            v = c1 - 1
            cumv = plsc.load_gather(cumb, [v * 16 + iota])
            l = plsc.all_reduce_population_count(cumv <= t)
            b_r = v * 16 + l - 1
            prefb = plsc.load_gather(cumb, [b_r])
            bvec = jnp.where(iota == rk, b_r, bvec)
            tresv = jnp.where(iota == rk, t - prefb, tresv)

        m9 = iota < K
        plsc.store_scatter(flag, [bvec], iota + 1, mask=m9)
        slotv = plsc.load_gather(flag, [bvec])

        def epass(i4, nf):
            i = i4 * 4
            parts = []
            for dd in range(4):
                bi = binb[pl.ds((i + dd) * 16, 16)]
                f = plsc.load_gather(flag, [bi])
                plsc.store_scatter(hist, [bi], zf)
                m = f > 0
                v = vals[pl.ds((i + dd) * 16, 16)]
                packed = ((i + dd) * 16 + iota) | (f << 12)
                parts.append((m, v, packed, jnp.sum(m.astype(jnp.int32))))
            off = nf
            for (m, v, packed, cnt) in parts:
                off = jnp.minimum(off, 496)
                plsc.store_compressed(fv.at[pl.ds(off, 16)], v, mask=m)
                plsc.store_compressed(fi.at[pl.ds(off, 16)], packed, mask=m)
                off = off + cnt
            return jnp.minimum(off, 496)
        nf = lax.fori_loop(0, NV // 4, epass, jnp.int32(0))
        plsc.store_scatter(flag, [bvec], zi, mask=m9)

        selv = zi
        nvf = (nf + 15) >> 4
        for rk in range(K):
            slot = jnp.sum(jnp.where(iota == rk, slotv, 0))
            tres = jnp.sum(jnp.where(iota == rk, tresv, 0.0)).astype(jnp.int32)

            def fscan(u, nb):
                fvv = fv[pl.ds(u * 16, 16)]
                fpp = fi[pl.ds(u * 16, 16)]
                m = ((fpp >> 12) == slot) & ((u * 16 + iota) < nf)
                plsc.store_compressed(tbv.at[pl.ds(nb, 16)], fvv, mask=m)
                plsc.store_compressed(tbi.at[pl.ds(nb, 16)], fpp & 0xFFF,
                                      mask=m)
                nb = nb + jnp.sum(m.astype(jnp.int32))
                return jnp.minimum(nb, 240)
            nb = lax.fori_loop(0, nvf, fscan, jnp.int32(0))

            def path_a(_):
                tv = tbv[pl.ds(0, 16)]
                ti = tbi[pl.ds(0, 16)]
                lm = iota < nb
                sk, sv, _ = plsc.sort_key_val(tv, ti, mask=lm)
                vstar = jnp.sum(jnp.where(iota == tres, sk, 0.0))
                cl = plsc.all_reduce_population_count((tv < vstar) & lm)
                tiem = (tv == vstar) & lm
                si, _, _ = plsc.sort_key_val(ti, ti, mask=tiem)
                pick = jnp.sum(jnp.where(iota == tres - cl, si, 0))
                return jnp.full((16,), pick, jnp.int32)

            def path_b(_):
                nbv = (nb + 15) >> 4

                def skey(x):
                    s = plsc.bitcast(x, jnp.int32)
                    return plsc.bitcast(
                        jnp.where(s < 0, ~s, s | jnp.int32(-2147483648)),
                        jnp.uint32)

                def bit_step(it, lo):
                    cand = lo | (jnp.uint32(1) << (31 - it).astype(jnp.uint32))

                    def cnt_scan(u, c):
                        kv = skey(tbv[pl.ds(u * 16, 16)])
                        m = (kv < cand) & ((u * 16 + iota) < nb)
                        return c + jnp.sum(m.astype(jnp.int32))
                    cnt = lax.fori_loop(0, nbv, cnt_scan, jnp.int32(0))
                    return jnp.where(cnt <= tres, cand, lo)
                lo = lax.fori_loop(0, 32, bit_step, jnp.uint32(0))
                si32 = plsc.bitcast(jnp.full((16,), lo, jnp.uint32), jnp.int32)
                vbits = jnp.where(si32 < 0, si32 & jnp.int32(2147483647), ~si32)
                vstar = plsc.bitcast(vbits, jnp.float32)

                def tie_scan(u, carry):
                    cl, nt = carry
                    tv = tbv[pl.ds(u * 16, 16)]
                    ti2 = tbi[pl.ds(u * 16, 16)]
                    valid = (u * 16 + iota) < nb
                    cl = cl + jnp.sum(((tv < vstar) & valid).astype(jnp.int32))
                    tm = (tv == vstar) & valid
                    plsc.store_compressed(tie.at[pl.ds(nt, 16)], ti2, mask=tm)
                    nt = jnp.minimum(nt + jnp.sum(tm.astype(jnp.int32)), 16)
                    return (cl, nt)
                cl, nt = lax.fori_loop(0, nbv, tie_scan,
                                       (jnp.int32(0), jnp.int32(0)))
                tv16 = tie[pl.ds(0, 16)]
                si, _, _ = plsc.sort_key_val(tv16, tv16, mask=iota < nt)
                pick = jnp.sum(jnp.where(iota == tres - cl, si, 0))
                return jnp.full((16,), pick, jnp.int32)

            pick = lax.cond(nb <= 16, path_a, path_b, 0)
            selv = jnp.where(iota == rk, pick, selv)

        ssel, _, _ = plsc.sort_key_val(selv, selv, mask=m9)
        ytidx = jnp.where(m9, (b * K + iota) * HW + ssel, 0)
        pltpu.async_copy(yt_hbm.at[ytidx], ytrows, sem).wait()
        for cb in range(C // 16):
            a = ytrows[0, pl.ds(cb * 16, 16)]
            for k in range(1, K):
                a = a + ytrows[k, pl.ds(cb * 16, 16)]
            accb[pl.ds(cb * 16, 16)] = a
        pltpu.sync_copy(accb, acc_hbm.at[row])
        return r

    def batch_body(b, c):
        def rloop(r, c2):
            row_body(b, r)
            return c2
        lax.fori_loop(0, RPW, rloop, 0)
        return c
    lax.fori_loop(0, B, batch_body, 0)


def _ep_body(acc_ref, bias_ref, x_ref, g_ref, out_ref):
    a = acc_ref[0]
    t = a + bias_ref[0][None, :]
    r = jnp.maximum(t, 0.0)
    rt = r.T
    out_ref[0] = g_ref[0, 0] * rt + x_ref[0]


def _epilogue(acc, bias, x3, gamma):
    return pl.pallas_call(
        _ep_body,
        grid=(B, NRB),
        in_specs=[
            pl.BlockSpec((1, RB, C), lambda b, j: (b * NRB + j, 0, 0)),
            pl.BlockSpec((1, C), lambda b, j: (0, 0)),
            pl.BlockSpec((1, C, RB), lambda b, j: (b, 0, j)),
            pl.BlockSpec((1, 1), lambda b, j: (0, 0)),
        ],
        out_specs=pl.BlockSpec((1, C, RB), lambda b, j: (b, 0, j)),
        out_shape=jax.ShapeDtypeStruct((B, C, HW), jnp.float32),
    )(acc, bias, x3, gamma)


def kernel(x, weight, bias, gamma):
    q = x.reshape(B, C, HW)
    en = _compute_en(q)
    yt = _compute_yt(q, jnp.transpose(weight, (2, 0, 1)))
    acc = _make_sc_select()(en.reshape(B * HW, HW),
                            yt.reshape(B * K * HW, C))
    out = _epilogue(acc.reshape(B * NRB, RB, C), bias.reshape(1, C), q,
                    gamma.reshape(1, 1))
    return out.reshape(B, C, H, W)
